# Initial kernel scaffold; baseline (speedup 1.0000x reference)
import functools

import jax
import jax.numpy as jnp
from jax import lax
from jax.experimental import pallas as pl
from jax.experimental.pallas import tpu as pltpu
from jax.experimental.pallas import tpu_sc as plsc

N = 10000
D = 128
G = 128
E = 320000
ALPHA = 0.1

NP = 10240
NC, NS = 2, 16
NW = NC * NS
CH = 128
K = 79
EP = NW * K * CH
EROWS = EP // CH
DRAIN = NP // NS

_mesh = plsc.VectorSubcoreMesh(core_axis_name="c", subcore_axis_name="s")


@functools.partial(
    pl.kernel,
    out_type=jax.ShapeDtypeStruct((NC, NP, 16), jnp.float32),
    mesh=_mesh,
    scratch_types=[
        pltpu.VMEM((K, CH), jnp.int32),
        pltpu.VMEM((CH, 16), jnp.float32),
        pltpu.VMEM_SHARED((NP, 16), jnp.float32),
    ],
)
def _sc_hist(dst_hbm, zeros_hbm, ones_hbm, out_hbm, idx_v, ones_v, acc_sh):
    cid = lax.axis_index("c")
    sid = lax.axis_index("s")
    wid = cid * NS + sid
    pltpu.sync_copy(zeros_hbm.at[pl.ds(sid * DRAIN, DRAIN)],
                    acc_sh.at[pl.ds(sid * DRAIN, DRAIN)])
    pltpu.sync_copy(ones_hbm, ones_v)
    pltpu.sync_copy(dst_hbm.at[pl.ds(wid * K, K)], idx_v)
    plsc.subcore_barrier()

    @pl.loop(0, K)
    def _(j):
        pltpu.sync_copy(ones_v, acc_sh.at[idx_v.at[j]], add=True)

    plsc.subcore_barrier()
    pltpu.sync_copy(acc_sh.at[pl.ds(sid * DRAIN, DRAIN)],
                    out_hbm.at[cid, pl.ds(sid * DRAIN, DRAIN)])


@functools.partial(
    pl.kernel,
    out_type=jax.ShapeDtypeStruct((NC, NP, D), jnp.float32),
    mesh=_mesh,
    scratch_types=[
        pltpu.VMEM((K, CH), jnp.int32),
        pltpu.VMEM((K, CH), jnp.int32),
        pltpu.VMEM((CH, D), jnp.float32),
        pltpu.VMEM_SHARED((NP, D), jnp.float32),
        pltpu.SemaphoreType.DMA,
    ],
)
def _sc_propagate(g_hbm, src_hbm, dst_hbm, zeros_hbm, out_hbm,
                  srcv, dstv, rows, acc_sh, sem):
    cid = lax.axis_index("c")
    sid = lax.axis_index("s")
    wid = cid * NS + sid
    pltpu.sync_copy(zeros_hbm.at[pl.ds(sid * DRAIN, DRAIN)],
                    acc_sh.at[pl.ds(sid * DRAIN, DRAIN)])
    pltpu.sync_copy(src_hbm.at[pl.ds(wid * K, K)], srcv)
    pltpu.sync_copy(dst_hbm.at[pl.ds(wid * K, K)], dstv)
    plsc.subcore_barrier()

    @pl.loop(0, K)
    def _(j):
        pltpu.async_copy(g_hbm.at[srcv.at[j]], rows, sem).wait()
        pltpu.sync_copy(rows, acc_sh.at[dstv.at[j]], add=True)

    plsc.subcore_barrier()
    pltpu.sync_copy(acc_sh.at[pl.ds(sid * DRAIN, DRAIN)],
                    out_hbm.at[cid, pl.ds(sid * DRAIN, DRAIN)])


_BLK = 1024


def _prep_body(x_ref, w_ref, b_ref, deg_ref, g_ref, cself_ref, dinv_ref):
    i = pl.program_id(0)
    h = lax.dot_general(x_ref[...], w_ref[...], (((1,), (0,)), ((), ())),
                        precision=lax.Precision.HIGHEST,
                        preferred_element_type=jnp.float32) + b_ref[...]
    h = h * lax.logistic(h)
    nrm = jnp.sqrt(jnp.sum(h * h, axis=1, keepdims=True))
    hp = h / jnp.maximum(nrm, 1e-12) * 1.8
    d = deg_ref[...]
    deg = d[0, :, 0:1] + d[1, :, 0:1] + 1.0
    dinv = lax.rsqrt(deg)
    row = i * _BLK + lax.broadcasted_iota(jnp.int32, (_BLK, 1), 0)
    mask = row < N
    g_ref[...] = jnp.where(mask, dinv * hp, 0.0)
    cself_ref[...] = jnp.where(mask, ((1.0 - ALPHA) / deg + ALPHA) * hp, 0.0)
    dinv_ref[...] = dinv


_prep = pl.pallas_call(
    _prep_body,
    grid=(NP // _BLK,),
    in_specs=[
        pl.BlockSpec((_BLK, D), lambda i: (i, 0)),
        pl.BlockSpec((D, D), lambda i: (0, 0)),
        pl.BlockSpec((1, D), lambda i: (0, 0)),
        pl.BlockSpec((NC, _BLK, 16), lambda i: (0, i, 0)),
    ],
    out_specs=[
        pl.BlockSpec((_BLK, D), lambda i: (i, 0)),
        pl.BlockSpec((_BLK, D), lambda i: (i, 0)),
        pl.BlockSpec((_BLK, 1), lambda i: (i, 0)),
    ],
    out_shape=[
        jax.ShapeDtypeStruct((NP, D), jnp.float32),
        jax.ShapeDtypeStruct((NP, D), jnp.float32),
        jax.ShapeDtypeStruct((NP, 1), jnp.float32),
    ],
)


def _comb_body(s_ref, cself_ref, dinv_ref, batch_ref, hout_ref, xg_ref):
    i = pl.program_id(0)
    s = s_ref[...]
    hout = (1.0 - ALPHA) * dinv_ref[...] * (s[0] + s[1]) + cself_ref[...]
    hout_ref[...] = hout
    oh = (batch_ref[...] ==
          lax.broadcasted_iota(jnp.int32, (_BLK, G), 1)).astype(jnp.float32)

    @pl.when(i == 0)
    def _():
        xg_ref[...] = jnp.zeros_like(xg_ref)

    xg_ref[...] += lax.dot_general(oh, hout, (((0,), (0,)), ((), ())),
                                   precision=lax.Precision.HIGHEST,
                                   preferred_element_type=jnp.float32)


_comb = pl.pallas_call(
    _comb_body,
    grid=(NP // _BLK,),
    in_specs=[
        pl.BlockSpec((NC, _BLK, D), lambda i: (0, i, 0)),
        pl.BlockSpec((_BLK, D), lambda i: (i, 0)),
        pl.BlockSpec((_BLK, 1), lambda i: (i, 0)),
        pl.BlockSpec((_BLK, 1), lambda i: (i, 0)),
    ],
    out_specs=[
        pl.BlockSpec((_BLK, D), lambda i: (i, 0)),
        pl.BlockSpec((G, D), lambda i: (0, 0)),
    ],
    out_shape=[
        jax.ShapeDtypeStruct((NP, D), jnp.float32),
        jax.ShapeDtypeStruct((G, D), jnp.float32),
    ],
)


def kernel(x, edge_index, batch, W, b):
    src = edge_index[0]
    dst = edge_index[1]
    pad = jnp.full((EP - E,), N, dtype=jnp.int32)
    src2d = jnp.concatenate([src, pad]).reshape(EROWS, CH)
    dst2d = jnp.concatenate([dst, pad]).reshape(EROWS, CH)
    x_pad = jnp.pad(x, ((0, NP - N), (0, 0)))
    batch2d = jnp.pad(batch, (0, NP - N)).reshape(NP, 1)
    zeros16 = jnp.zeros((NP, 16), jnp.float32)
    ones16 = jnp.ones((CH, 16), jnp.float32)
    zerosD = jnp.zeros((NP, D), jnp.float32)

    degp = _sc_hist(dst2d, zeros16, ones16)
    g, cself, dinv = _prep(x_pad, W, b.reshape(1, D), degp)
    s_part = _sc_propagate(g, src2d, dst2d, zerosD)
    hout_pad, xg = _comb(s_part, cself, dinv, batch2d)
    return hout_pad[:N], xg


# speedup vs baseline: 13.6145x; 13.6145x over previous
import functools

import jax
import jax.numpy as jnp
from jax import lax
from jax.experimental import pallas as pl
from jax.experimental.pallas import tpu as pltpu
from jax.experimental.pallas import tpu_sc as plsc

N = 10000
D = 128
G = 128
E = 320000
ALPHA = 0.1

NP = 10240
NC, NS = 2, 16
NW = NC * NS
CH = 128
K = 80
EP = NW * K * CH
EROWS = EP // CH
DRAIN = NP // NS

_mesh = plsc.VectorSubcoreMesh(core_axis_name="c", subcore_axis_name="s")


@functools.partial(
    pl.kernel,
    out_type=jax.ShapeDtypeStruct((NC, NP, 16), jnp.float32),
    mesh=_mesh,
    scratch_types=[
        pltpu.VMEM((K, CH), jnp.int32),
        pltpu.VMEM((CH, 16), jnp.float32),
        pltpu.VMEM_SHARED((NP, 16), jnp.float32),
    ],
)
def _sc_hist(dst_hbm, zeros_hbm, ones_hbm, out_hbm, idx_v, ones_v, acc_sh):
    cid = lax.axis_index("c")
    sid = lax.axis_index("s")
    wid = cid * NS + sid
    pltpu.sync_copy(zeros_hbm.at[pl.ds(sid * DRAIN, DRAIN)],
                    acc_sh.at[pl.ds(sid * DRAIN, DRAIN)])
    pltpu.sync_copy(ones_hbm, ones_v)
    pltpu.sync_copy(dst_hbm.at[pl.ds(wid * K, K)], idx_v)
    plsc.subcore_barrier()

    @pl.loop(0, K)
    def _(j):
        pltpu.sync_copy(ones_v, acc_sh.at[idx_v.at[j]], add=True)

    plsc.subcore_barrier()
    pltpu.sync_copy(acc_sh.at[pl.ds(sid * DRAIN, DRAIN)],
                    out_hbm.at[cid, pl.ds(sid * DRAIN, DRAIN)])


@functools.partial(
    pl.kernel,
    out_type=jax.ShapeDtypeStruct((NC, NP, D), jnp.float32),
    mesh=_mesh,
    scratch_types=[
        pltpu.VMEM((K, CH), jnp.int32),
        pltpu.VMEM((K, CH), jnp.int32),
        pltpu.VMEM((CH, D), jnp.float32),
        pltpu.VMEM_SHARED((NP, D), jnp.float32),
        pltpu.SemaphoreType.DMA,
    ],
)
def _sc_propagate(g_hbm, src_hbm, dst_hbm, zeros_hbm, out_hbm,
                  srcv, dstv, rows, acc_sh, sem):
    cid = lax.axis_index("c")
    sid = lax.axis_index("s")
    wid = cid * NS + sid
    pltpu.sync_copy(zeros_hbm.at[pl.ds(sid * DRAIN, DRAIN)],
                    acc_sh.at[pl.ds(sid * DRAIN, DRAIN)])
    pltpu.sync_copy(src_hbm.at[pl.ds(wid * K, K)], srcv)
    pltpu.sync_copy(dst_hbm.at[pl.ds(wid * K, K)], dstv)
    plsc.subcore_barrier()

    @pl.loop(0, K)
    def _(j):
        pltpu.async_copy(g_hbm.at[srcv.at[j]], rows, sem).wait()
        pltpu.sync_copy(rows, acc_sh.at[dstv.at[j]], add=True)

    plsc.subcore_barrier()
    pltpu.sync_copy(acc_sh.at[pl.ds(sid * DRAIN, DRAIN)],
                    out_hbm.at[cid, pl.ds(sid * DRAIN, DRAIN)])


_BLK = 1024


def _prep_body(x_ref, w_ref, b_ref, deg_ref, g_ref, cself_ref, dinv_ref):
    i = pl.program_id(0)
    h = lax.dot_general(x_ref[...], w_ref[...], (((1,), (0,)), ((), ())),
                        precision=lax.Precision.HIGHEST,
                        preferred_element_type=jnp.float32) + b_ref[...]
    h = h * lax.logistic(h)
    nrm = jnp.sqrt(jnp.sum(h * h, axis=1, keepdims=True))
    hp = h / jnp.maximum(nrm, 1e-12) * 1.8
    d = deg_ref[...]
    deg = d[0, :, 0:1] + d[1, :, 0:1] + 1.0
    dinv = lax.rsqrt(deg)
    row = i * _BLK + lax.broadcasted_iota(jnp.int32, (_BLK, 1), 0)
    mask = row < N
    g_ref[...] = jnp.where(mask, dinv * hp, 0.0)
    cself_ref[...] = jnp.where(mask, ((1.0 - ALPHA) / deg + ALPHA) * hp, 0.0)
    dinv_ref[...] = dinv


_prep = pl.pallas_call(
    _prep_body,
    grid=(NP // _BLK,),
    in_specs=[
        pl.BlockSpec((_BLK, D), lambda i: (i, 0)),
        pl.BlockSpec((D, D), lambda i: (0, 0)),
        pl.BlockSpec((1, D), lambda i: (0, 0)),
        pl.BlockSpec((NC, _BLK, 16), lambda i: (0, i, 0)),
    ],
    out_specs=[
        pl.BlockSpec((_BLK, D), lambda i: (i, 0)),
        pl.BlockSpec((_BLK, D), lambda i: (i, 0)),
        pl.BlockSpec((_BLK, 1), lambda i: (i, 0)),
    ],
    out_shape=[
        jax.ShapeDtypeStruct((NP, D), jnp.float32),
        jax.ShapeDtypeStruct((NP, D), jnp.float32),
        jax.ShapeDtypeStruct((NP, 1), jnp.float32),
    ],
)


def _comb_body(s_ref, cself_ref, dinv_ref, batch_ref, hout_ref, xg_ref):
    i = pl.program_id(0)
    s = s_ref[...]
    hout = (1.0 - ALPHA) * dinv_ref[...] * (s[0] + s[1]) + cself_ref[...]
    hout_ref[...] = hout
    oh = (batch_ref[...] ==
          lax.broadcasted_iota(jnp.int32, (_BLK, G), 1)).astype(jnp.float32)

    @pl.when(i == 0)
    def _():
        xg_ref[...] = jnp.zeros_like(xg_ref)

    xg_ref[...] += lax.dot_general(oh, hout, (((0,), (0,)), ((), ())),
                                   precision=lax.Precision.HIGHEST,
                                   preferred_element_type=jnp.float32)


_comb = pl.pallas_call(
    _comb_body,
    grid=(NP // _BLK,),
    in_specs=[
        pl.BlockSpec((NC, _BLK, D), lambda i: (0, i, 0)),
        pl.BlockSpec((_BLK, D), lambda i: (i, 0)),
        pl.BlockSpec((_BLK, 1), lambda i: (i, 0)),
        pl.BlockSpec((_BLK, 1), lambda i: (i, 0)),
    ],
    out_specs=[
        pl.BlockSpec((_BLK, D), lambda i: (i, 0)),
        pl.BlockSpec((G, D), lambda i: (0, 0)),
    ],
    out_shape=[
        jax.ShapeDtypeStruct((NP, D), jnp.float32),
        jax.ShapeDtypeStruct((G, D), jnp.float32),
    ],
)


def kernel(x, edge_index, batch, W, b):
    src = edge_index[0]
    dst = edge_index[1]
    pad = jnp.full((EP - E,), N, dtype=jnp.int32)
    src2d = jnp.concatenate([src, pad]).reshape(EROWS, CH)
    dst2d = jnp.concatenate([dst, pad]).reshape(EROWS, CH)
    x_pad = jnp.pad(x, ((0, NP - N), (0, 0)))
    batch2d = jnp.pad(batch, (0, NP - N)).reshape(NP, 1)
    zeros16 = jnp.zeros((NP, 16), jnp.float32)
    ones16 = jnp.ones((CH, 16), jnp.float32)
    zerosD = jnp.zeros((NP, D), jnp.float32)

    degp = _sc_hist(dst2d, zeros16, ones16)
    g, cself, dinv = _prep(x_pad, W, b.reshape(1, D), degp)
    s_part = _sc_propagate(g, src2d, dst2d, zerosD)
    hout_pad, xg = _comb(s_part, cself, dinv, batch2d)
    return hout_pad[:N], xg
